# Initial kernel scaffold; baseline (speedup 1.0000x reference)
import jax
import jax.numpy as jnp
from jax.experimental import pallas as pl


def kernel(means3d, opacities, features, scales, rotations, empty_scalar):
    raise NotImplementedError("write your pallas kernel here")


# speedup vs baseline: 6.7634x; 6.7634x over previous
import functools

import jax
import jax.numpy as jnp
from jax.experimental import pallas as pl

VOL_LO = (-40.0, -40.0, -1.0)
VOL_HI = (40.0, 40.0, 5.4)
VOXEL_SIZE = 0.4
NUM_CLASSES = 18
EMPTY_LABEL = 17
GRID_SHAPE = (200, 200, 16)
GP = 640
TILE = 1024
FCOLS = 32


def _prologue_body(in_ref, out_ref):
    def row(a):
        return in_ref[a:a + 1, :]

    qw, qx, qy, qz = row(0), row(1), row(2), row(3)
    n = jnp.sqrt(qw * qw + qx * qx + qy * qy + qz * qz)
    w, x, y, z = qw / n, qx / n, qy / n, qz / n

    r00 = 1.0 - 2.0 * (y * y + z * z)
    r01 = 2.0 * (x * y - w * z)
    r02 = 2.0 * (x * z + w * y)
    r10 = 2.0 * (x * y + w * z)
    r11 = 1.0 - 2.0 * (x * x + z * z)
    r12 = 2.0 * (y * z - w * x)
    r20 = 2.0 * (x * z - w * y)
    r21 = 2.0 * (y * z + w * x)
    r22 = 1.0 - 2.0 * (x * x + y * y)

    s0, s1, s2 = row(4), row(5), row(6)
    l00, l01, l02 = r00 * s0, r01 * s1, r02 * s2
    l10, l11, l12 = r10 * s0, r11 * s1, r12 * s2
    l20, l21, l22 = r20 * s0, r21 * s1, r22 * s2
    c00 = l00 * l00 + l01 * l01 + l02 * l02
    c01 = l00 * l10 + l01 * l11 + l02 * l12
    c02 = l00 * l20 + l01 * l21 + l02 * l22
    c11 = l10 * l10 + l11 * l11 + l12 * l12
    c12 = l10 * l20 + l11 * l21 + l12 * l22
    c22 = l20 * l20 + l21 * l21 + l22 * l22

    det = (c00 * (c11 * c22 - c12 * c12)
           - c01 * (c01 * c22 - c12 * c02)
           + c02 * (c01 * c12 - c11 * c02))
    inv_det = 1.0 / det
    ia00 = (c11 * c22 - c12 * c12) * inv_det
    ia11 = (c00 * c22 - c02 * c02) * inv_det
    ia22 = (c00 * c11 - c01 * c01) * inv_det
    ia01 = (c02 * c12 - c01 * c22) * inv_det
    ia02 = (c01 * c12 - c02 * c11) * inv_det
    ia12 = (c02 * c01 - c00 * c12) * inv_det

    mx, my, mz = row(7), row(8), row(9)
    opac = row(10)

    keep = None
    bounds = []
    for a, (m, caa) in enumerate(((mx, c00), (my, c11), (mz, c22))):
        sig = jnp.sqrt(caa)
        blo = jnp.clip(m - 3.0 * sig, VOL_LO[a], VOL_HI[a])
        bhi = jnp.clip(m + 3.0 * sig, VOL_LO[a], VOL_HI[a])
        lo = jnp.floor((blo - VOL_LO[a]) / VOXEL_SIZE)
        hi = jnp.floor((bhi - VOL_LO[a]) / VOXEL_SIZE)
        bounds.append((lo, hi))
        any_gt = (blo > VOL_LO[a]) | (bhi > VOL_LO[a])
        any_lt = (blo < VOL_HI[a]) | (bhi < VOL_HI[a])
        ka = any_gt & any_lt
        keep = ka if keep is None else (keep & ka)
    opac_eff = jnp.where(keep, opac, 0.0)

    rows_out = [ia00, ia11, ia22, ia01, ia02, ia12, mx, my, mz,
                bounds[0][0], bounds[1][0], bounds[2][0],
                bounds[0][1], bounds[1][1], bounds[2][1], opac_eff]
    out_ref[...] = jnp.concatenate(rows_out, axis=0)


def _splat_body(coef_ref, feat_ref, out_ref):
    t = pl.program_id(0)
    H, W, D = GRID_SHAPE

    def crow(a):
        return coef_ref[a:a + 1, :]

    v = t * TILE + jax.lax.broadcasted_iota(jnp.int32, (TILE, 1), 0)
    i = v // (W * D)
    rem = v - i * (W * D)
    j = rem // D
    k = rem - j * D
    fi = i.astype(jnp.float32)
    fj = j.astype(jnp.float32)
    fk = k.astype(jnp.float32)
    xf = (fi + 0.5) * VOXEL_SIZE + VOL_LO[0]
    yf = (fj + 0.5) * VOXEL_SIZE + VOL_LO[1]
    zf = (fk + 0.5) * VOXEL_SIZE + VOL_LO[2]

    dx = xf - crow(6)
    dy = yf - crow(7)
    dz = zf - crow(8)
    r1 = crow(0) * dx + crow(3) * dy + crow(4) * dz
    r2 = crow(3) * dx + crow(1) * dy + crow(5) * dz
    r3 = crow(4) * dx + crow(5) * dy + crow(2) * dz
    maha = dx * r1 + dy * r2 + dz * r3

    d = crow(15) * jnp.exp(-0.5 * maha)
    mask = ((fi >= crow(9)) & (fi <= crow(12))
            & (fj >= crow(10)) & (fj <= crow(13))
            & (fk >= crow(11)) & (fk <= crow(14)))
    d = jnp.where(mask, d, 0.0)

    acc = jnp.dot(d, feat_ref[...], preferred_element_type=jnp.float32,
                  precision=jax.lax.Precision.HIGHEST)
    dens = acc[:, 0:1]
    norm = acc / jnp.maximum(dens, 1e-6)
    out_ref[...] = jnp.concatenate([dens, norm[:, 1:]], axis=1)


@jax.jit
def kernel(means3d, opacities, features, scales, rotations, empty_scalar):
    H, W, D = GRID_SHAPE
    V = H * W * D
    N = means3d.shape[0]
    center = jnp.array([(a + b) / 2.0 for a, b in zip(VOL_LO, VOL_HI)],
                       dtype=jnp.float32)
    ranges = jnp.array([b - a for a, b in zip(VOL_LO, VOL_HI)],
                       dtype=jnp.float32)

    packed = jnp.zeros((16, GP), dtype=jnp.float32)
    packed = packed.at[0, :].set(1.0)
    packed = packed.at[4:7, :].set(1.0)
    packed = packed.at[0:4, :N].set(rotations.astype(jnp.float32).T)
    packed = packed.at[4:7, :N].set(scales.astype(jnp.float32).T)
    packed = packed.at[7:10, :N].set(means3d.astype(jnp.float32).T)
    packed = packed.at[10, :N].set(opacities.astype(jnp.float32)[:, 0])
    packed = packed.at[4:7, N].set(ranges)
    packed = packed.at[7:10, N].set(center)
    packed = packed.at[10, N].set(1.0)

    coefs = pl.pallas_call(
        _prologue_body,
        out_shape=jax.ShapeDtypeStruct((16, GP), jnp.float32),
    )(packed)

    feat = jnp.zeros((GP, FCOLS), dtype=jnp.float32)
    feat = feat.at[:, 0].set(1.0)
    feat = feat.at[:N, 1:NUM_CLASSES].set(features.astype(jnp.float32))
    feat = feat.at[N, 1 + EMPTY_LABEL].set(empty_scalar[0])

    out = pl.pallas_call(
        _splat_body,
        grid=(V // TILE,),
        in_specs=[
            pl.BlockSpec((16, GP), lambda t: (0, 0)),
            pl.BlockSpec((GP, FCOLS), lambda t: (0, 0)),
        ],
        out_specs=pl.BlockSpec((TILE, FCOLS), lambda t: (t, 0)),
        out_shape=jax.ShapeDtypeStruct((V, FCOLS), jnp.float32),
    )(coefs, feat)

    grid_density = out[:, 0:1].reshape(H, W, D, 1)
    grid_feats = out[:, 1:1 + NUM_CLASSES].reshape(H, W, D, NUM_CLASSES)
    return grid_density, grid_feats


# speedup vs baseline: 54.2895x; 8.0269x over previous
import functools

import jax
import jax.numpy as jnp
from jax import lax
from jax.experimental import pallas as pl
from jax.experimental.pallas import tpu as pltpu
from jax.experimental.pallas import tpu_sc as plsc

VOL_LO = (-40.0, -40.0, -1.0)
VOL_HI = (40.0, 40.0, 5.4)
VOXEL_SIZE = 0.4
NUM_CLASSES = 18
EMPTY_LABEL = 17
GRID_SHAPE = (200, 200, 16)
GP = 512
NW = 32
CH = 1 + NUM_CLASSES
ROWW = CH * 200 * 16
PC = 48


def _prologue_body(in_ref, out_ref):
    def row(a):
        return in_ref[a:a + 1, :]

    qw, qx, qy, qz = row(0), row(1), row(2), row(3)
    n = jnp.sqrt(qw * qw + qx * qx + qy * qy + qz * qz)
    w, x, y, z = qw / n, qx / n, qy / n, qz / n

    r00 = 1.0 - 2.0 * (y * y + z * z)
    r01 = 2.0 * (x * y - w * z)
    r02 = 2.0 * (x * z + w * y)
    r10 = 2.0 * (x * y + w * z)
    r11 = 1.0 - 2.0 * (x * x + z * z)
    r12 = 2.0 * (y * z - w * x)
    r20 = 2.0 * (x * z - w * y)
    r21 = 2.0 * (y * z + w * x)
    r22 = 1.0 - 2.0 * (x * x + y * y)

    s0, s1, s2 = row(4), row(5), row(6)
    l00, l01, l02 = r00 * s0, r01 * s1, r02 * s2
    l10, l11, l12 = r10 * s0, r11 * s1, r12 * s2
    l20, l21, l22 = r20 * s0, r21 * s1, r22 * s2
    c00 = l00 * l00 + l01 * l01 + l02 * l02
    c01 = l00 * l10 + l01 * l11 + l02 * l12
    c02 = l00 * l20 + l01 * l21 + l02 * l22
    c11 = l10 * l10 + l11 * l11 + l12 * l12
    c12 = l10 * l20 + l11 * l21 + l12 * l22
    c22 = l20 * l20 + l21 * l21 + l22 * l22

    det = (c00 * (c11 * c22 - c12 * c12)
           - c01 * (c01 * c22 - c12 * c02)
           + c02 * (c01 * c12 - c11 * c02))
    inv_det = 1.0 / det
    ia00 = (c11 * c22 - c12 * c12) * inv_det
    ia11 = (c00 * c22 - c02 * c02) * inv_det
    ia22 = (c00 * c11 - c01 * c01) * inv_det
    ia01 = (c02 * c12 - c01 * c22) * inv_det
    ia02 = (c01 * c12 - c02 * c11) * inv_det
    ia12 = (c02 * c01 - c00 * c12) * inv_det

    mx, my, mz = row(7), row(8), row(9)
    opac = row(10)

    keep = None
    bounds = []
    for a, (m, caa) in enumerate(((mx, c00), (my, c11), (mz, c22))):
        sig = jnp.sqrt(caa)
        blo = jnp.clip(m - 3.0 * sig, VOL_LO[a], VOL_HI[a])
        bhi = jnp.clip(m + 3.0 * sig, VOL_LO[a], VOL_HI[a])
        lo = jnp.floor((blo - VOL_LO[a]) / VOXEL_SIZE)
        hi = jnp.floor((bhi - VOL_LO[a]) / VOXEL_SIZE)
        bounds.append((lo, hi))
        any_gt = (blo > VOL_LO[a]) | (bhi > VOL_LO[a])
        any_lt = (blo < VOL_HI[a]) | (bhi < VOL_HI[a])
        ka = any_gt & any_lt
        keep = ka if keep is None else (keep & ka)
    opac_eff = jnp.where(keep, opac, 0.0)

    rows_out = [ia00, ia11, ia22, ia01, ia02, ia12, mx, my, mz,
                bounds[0][0], bounds[1][0], bounds[2][0],
                bounds[0][1], bounds[1][1], bounds[2][1], opac_eff]
    out_ref[...] = jnp.concatenate(rows_out, axis=0)


def _sc_body(params_hbm, out_hbm, params_v, buf, ey_v):
    H, W, D = GRID_SHAPE
    center = tuple((a + b) / 2.0 for a, b in zip(VOL_LO, VOL_HI))
    inv_r2 = tuple(1.0 / (b - a) ** 2 for a, b in zip(VOL_LO, VOL_HI))
    wid = lax.axis_index("s") * 2 + lax.axis_index("c")
    pltpu.sync_copy(params_hbm, params_v)

    lane = lax.iota(jnp.int32, 16)
    lanef = lane.astype(jnp.float32)
    zf = (lanef + 0.5) * VOXEL_SIZE + VOL_LO[2]
    dze = zf - center[2]
    ez = jnp.exp(-0.5 * dze * dze * inv_r2[2])
    es = params_v[pl.ds(32, 16)][14]
    esv = jnp.full((16,), es)

    rowsf = (jnp.full((16,), wid) + 32 * lane).astype(jnp.float32)
    xrow = (rowsf + 0.5) * VOXEL_SIZE + VOL_LO[0]
    dxe = xrow - center[0]
    exrow = jnp.exp(-0.5 * dxe * dxe * inv_r2[0])
    for chunk in range(13):
        jv = (jnp.full((16,), chunk * 16) + lane).astype(jnp.float32)
        yv = (jv + 0.5) * VOXEL_SIZE + VOL_LO[1]
        dye = yv - center[1]
        ey_v[pl.ds(chunk * 16, 16)] = jnp.exp(-0.5 * dye * dye * inv_r2[1])

    zero16 = jnp.zeros((16,), jnp.float32)

    for t in range(7):
        i = wid + 32 * t

        @pl.when(i < H)
        def _row():
            fi = i.astype(jnp.float32)
            xi = (fi + 0.5) * VOXEL_SIZE + VOL_LO[0]
            ex_i = exrow[t]

            def init_body(j, c):
                sy = ex_i * ey_v[pl.ds(j, 16)][0]
                bg = jnp.full((16,), sy) * ez
                base = j * 16
                buf[pl.ds(base, 16)] = bg
                for ch in range(1, CH - 1):
                    buf[pl.ds(ch * 3200 + base, 16)] = zero16
                buf[pl.ds((CH - 1) * 3200 + base, 16)] = bg * esv
                return c

            lax.fori_loop(0, W, init_body, 0)

            def g_body(g, c):
                pv0 = params_v[pl.ds(g * PC, 16)]
                lo0 = pv0[9]
                hi0 = pv0[12]
                op = pv0[15]
                pred = (fi >= lo0) & (fi <= hi0) & (op > 0.0)

                @pl.when(pred)
                def _splat():
                    ia00 = pv0[0]
                    ia11 = pv0[1]
                    ia22 = pv0[2]
                    ia01 = pv0[3]
                    ia02 = pv0[4]
                    ia12 = pv0[5]
                    mx = pv0[6]
                    my = pv0[7]
                    mz = pv0[8]
                    lo1 = pv0[10]
                    lo2 = pv0[11]
                    hi1 = pv0[13]
                    hi2 = pv0[14]
                    pv1 = params_v[pl.ds(g * PC + 16, 16)]
                    pv2 = params_v[pl.ds(g * PC + 32, 16)]

                    dx = xi - mx
                    dzv = zf - jnp.full((16,), mz)
                    v0 = (jnp.full((16,), ia00 * dx * dx)
                          + (jnp.full((16,), ia22) * dzv
                             + jnp.full((16,), 2.0 * ia02 * dx)) * dzv)
                    c1 = 2.0 * ia01 * dx
                    km = ((lanef >= jnp.full((16,), lo2))
                          & (lanef <= jnp.full((16,), hi2)))
                    opv = jnp.where(km, jnp.full((16,), op), 0.0)
                    fv = [jnp.full((16,), pv1[cc]) if cc < 16
                          else jnp.full((16,), pv2[cc - 16])
                          for cc in range(NUM_CLASSES)]
                    j0 = lo1.astype(jnp.int32)
                    j1 = jnp.minimum(hi1, jnp.float32(W - 1)).astype(jnp.int32)

                    def j_body(j, cc):
                        jf = j.astype(jnp.float32)
                        yj = (jf + 0.5) * VOXEL_SIZE + VOL_LO[1]
                        dy = yj - my
                        s1 = dy * (ia11 * dy + c1)
                        s2 = 2.0 * ia12 * dy
                        maha = (v0 + jnp.full((16,), s1)
                                + jnp.full((16,), s2) * dzv)
                        d = opv * jnp.exp(-0.5 * maha)
                        base = j * 16
                        plsc.addupdate(buf.at[pl.ds(base, 16)], d)
                        for ch in range(NUM_CLASSES):
                            plsc.addupdate(
                                buf.at[pl.ds((ch + 1) * 3200 + base, 16)],
                                fv[ch] * d)
                        return cc

                    lax.fori_loop(j0, j1 + 1, j_body, 0)
                return c

            lax.fori_loop(0, GP, g_body, 0)

            def norm_body(j, c):
                base = j * 16
                dd = buf[pl.ds(base, 16)]
                r = 1.0 / jnp.maximum(dd, 1e-6)
                for ch in range(1, CH):
                    o = ch * 3200 + base
                    buf[pl.ds(o, 16)] = buf[pl.ds(o, 16)] * r
                return c

            lax.fori_loop(0, W, norm_body, 0)
            pltpu.sync_copy(buf, out_hbm.at[pl.ds(i * ROWW, ROWW)])


@jax.jit
def kernel(means3d, opacities, features, scales, rotations, empty_scalar):
    H, W, D = GRID_SHAPE
    V = H * W * D
    N = means3d.shape[0]

    packed = jnp.zeros((16, GP), dtype=jnp.float32)
    packed = packed.at[0:4, :N].set(rotations.astype(jnp.float32).T)
    packed = packed.at[4:7, :N].set(scales.astype(jnp.float32).T)
    packed = packed.at[7:10, :N].set(means3d.astype(jnp.float32).T)
    packed = packed.at[10, :N].set(opacities.astype(jnp.float32)[:, 0])

    coefs = pl.pallas_call(
        _prologue_body,
        out_shape=jax.ShapeDtypeStruct((16, GP), jnp.float32),
    )(packed)

    params = jnp.concatenate(
        [coefs.T,
         features.astype(jnp.float32),
         jnp.zeros((N, 1), dtype=jnp.float32),
         jnp.zeros((N, PC - 16 - NUM_CLASSES), dtype=jnp.float32)],
        axis=1)
    params = params.at[:, 46].set(empty_scalar[0])

    mesh = plsc.VectorSubcoreMesh(core_axis_name="c", subcore_axis_name="s")
    out = pl.kernel(
        _sc_body,
        mesh=mesh,
        out_type=jax.ShapeDtypeStruct((H * ROWW,), jnp.float32),
        scratch_types=[
            pltpu.VMEM((GP * PC,), jnp.float32),
            pltpu.VMEM((ROWW,), jnp.float32),
            pltpu.VMEM((224,), jnp.float32),
        ],
    )(params.reshape(-1))

    o3 = out.reshape(H, CH, W * D)
    grid_density = o3[:, 0, :].reshape(H, W, D, 1)
    grid_feats = o3[:, 1:, :].transpose(0, 2, 1).reshape(
        H, W, D, NUM_CLASSES)
    return grid_density, grid_feats


# speedup vs baseline: 57.5432x; 1.0599x over previous
import functools

import jax
import jax.numpy as jnp
from jax import lax
from jax.experimental import pallas as pl
from jax.experimental.pallas import tpu as pltpu
from jax.experimental.pallas import tpu_sc as plsc

VOL_LO = (-40.0, -40.0, -1.0)
VOL_HI = (40.0, 40.0, 5.4)
VOXEL_SIZE = 0.4
NUM_CLASSES = 18
EMPTY_LABEL = 17
GRID_SHAPE = (200, 200, 16)
GP = 512
NW = 32
CH = 1 + NUM_CLASSES
CHP = 24
PC = 48


def _prologue_body(in_ref, out_ref):
    def row(a):
        return in_ref[a:a + 1, :]

    qw, qx, qy, qz = row(0), row(1), row(2), row(3)
    n = jnp.sqrt(qw * qw + qx * qx + qy * qy + qz * qz)
    w, x, y, z = qw / n, qx / n, qy / n, qz / n

    r00 = 1.0 - 2.0 * (y * y + z * z)
    r01 = 2.0 * (x * y - w * z)
    r02 = 2.0 * (x * z + w * y)
    r10 = 2.0 * (x * y + w * z)
    r11 = 1.0 - 2.0 * (x * x + z * z)
    r12 = 2.0 * (y * z - w * x)
    r20 = 2.0 * (x * z - w * y)
    r21 = 2.0 * (y * z + w * x)
    r22 = 1.0 - 2.0 * (x * x + y * y)

    s0, s1, s2 = row(4), row(5), row(6)
    l00, l01, l02 = r00 * s0, r01 * s1, r02 * s2
    l10, l11, l12 = r10 * s0, r11 * s1, r12 * s2
    l20, l21, l22 = r20 * s0, r21 * s1, r22 * s2
    c00 = l00 * l00 + l01 * l01 + l02 * l02
    c01 = l00 * l10 + l01 * l11 + l02 * l12
    c02 = l00 * l20 + l01 * l21 + l02 * l22
    c11 = l10 * l10 + l11 * l11 + l12 * l12
    c12 = l10 * l20 + l11 * l21 + l12 * l22
    c22 = l20 * l20 + l21 * l21 + l22 * l22

    det = (c00 * (c11 * c22 - c12 * c12)
           - c01 * (c01 * c22 - c12 * c02)
           + c02 * (c01 * c12 - c11 * c02))
    inv_det = 1.0 / det
    ia00 = (c11 * c22 - c12 * c12) * inv_det
    ia11 = (c00 * c22 - c02 * c02) * inv_det
    ia22 = (c00 * c11 - c01 * c01) * inv_det
    ia01 = (c02 * c12 - c01 * c22) * inv_det
    ia02 = (c01 * c12 - c02 * c11) * inv_det
    ia12 = (c02 * c01 - c00 * c12) * inv_det

    mx, my, mz = row(7), row(8), row(9)
    opac = row(10)

    keep = None
    bounds = []
    for a, (m, caa) in enumerate(((mx, c00), (my, c11), (mz, c22))):
        sig = jnp.sqrt(caa)
        blo = jnp.clip(m - 3.0 * sig, VOL_LO[a], VOL_HI[a])
        bhi = jnp.clip(m + 3.0 * sig, VOL_LO[a], VOL_HI[a])
        lo = jnp.floor((blo - VOL_LO[a]) / VOXEL_SIZE)
        hi = jnp.floor((bhi - VOL_LO[a]) / VOXEL_SIZE)
        bounds.append((lo, hi))
        any_gt = (blo > VOL_LO[a]) | (bhi > VOL_LO[a])
        any_lt = (blo < VOL_HI[a]) | (bhi < VOL_HI[a])
        ka = any_gt & any_lt
        keep = ka if keep is None else (keep & ka)
    opac_eff = jnp.where(keep, opac, 0.0)

    rows_out = [ia00, ia11, ia22, ia01, ia02, ia12, mx, my, mz,
                bounds[0][0], bounds[1][0], bounds[2][0],
                bounds[0][1], bounds[1][1], bounds[2][1], opac_eff]
    out_ref[...] = jnp.concatenate(rows_out, axis=0)


def _sc_body(params_hbm, out_hbm, params_v, buf, ey_v):
    H, W, D = GRID_SHAPE
    center = tuple((a + b) / 2.0 for a, b in zip(VOL_LO, VOL_HI))
    inv_r2 = tuple(1.0 / (b - a) ** 2 for a, b in zip(VOL_LO, VOL_HI))
    wid = lax.axis_index("s") * 2 + lax.axis_index("c")
    pltpu.sync_copy(params_hbm, params_v)

    lane = lax.iota(jnp.int32, 16)
    lanef = lane.astype(jnp.float32)
    zf = (lanef + 0.5) * VOXEL_SIZE + VOL_LO[2]
    dze = zf - center[2]
    ez = jnp.exp(-0.5 * dze * dze * inv_r2[2])
    es = params_v[pl.ds(32, 16)][14]
    esv = jnp.full((16,), es)

    rowsf = (jnp.full((16,), wid) + 32 * lane).astype(jnp.float32)
    xrow = (rowsf + 0.5) * VOXEL_SIZE + VOL_LO[0]
    dxe = xrow - center[0]
    exrow = jnp.exp(-0.5 * dxe * dxe * inv_r2[0])
    for chunk in range(13):
        jv = (jnp.full((16,), chunk * 16) + lane).astype(jnp.float32)
        yv = (jv + 0.5) * VOXEL_SIZE + VOL_LO[1]
        dye = yv - center[1]
        ey_v[pl.ds(chunk * 16, 16)] = jnp.exp(-0.5 * dye * dye * inv_r2[1])

    zero16 = jnp.zeros((16,), jnp.float32)

    for t in range(7):
        i = wid + 32 * t

        @pl.when(i < H)
        def _row():
            fi = i.astype(jnp.float32)
            xi = (fi + 0.5) * VOXEL_SIZE + VOL_LO[0]
            ex_i = exrow[t]

            def init_body(j, c):
                sy = ex_i * ey_v[pl.ds(j, 16)][0]
                bg = jnp.full((16,), sy) * ez
                base = j * 16
                buf[0, pl.ds(base, 16)] = bg
                for ch in range(1, CH - 1):
                    buf[ch, pl.ds(base, 16)] = zero16
                buf[CH - 1, pl.ds(base, 16)] = bg * esv
                return c

            lax.fori_loop(0, W, init_body, 0)

            def g_body(g, c):
                pv0 = params_v[pl.ds(g * PC, 16)]
                lo0 = pv0[9]
                hi0 = pv0[12]
                op = pv0[15]
                pred = (fi >= lo0) & (fi <= hi0) & (op > 0.0)

                @pl.when(pred)
                def _splat():
                    ia00 = pv0[0]
                    ia11 = pv0[1]
                    ia22 = pv0[2]
                    ia01 = pv0[3]
                    ia02 = pv0[4]
                    ia12 = pv0[5]
                    mx = pv0[6]
                    my = pv0[7]
                    mz = pv0[8]
                    lo1 = pv0[10]
                    lo2 = pv0[11]
                    hi1 = pv0[13]
                    hi2 = pv0[14]
                    pv1 = params_v[pl.ds(g * PC + 16, 16)]
                    pv2 = params_v[pl.ds(g * PC + 32, 16)]

                    dx = xi - mx
                    dzv = zf - jnp.full((16,), mz)
                    v0 = (jnp.full((16,), ia00 * dx * dx)
                          + (jnp.full((16,), ia22) * dzv
                             + jnp.full((16,), 2.0 * ia02 * dx)) * dzv)
                    c1 = 2.0 * ia01 * dx
                    km = ((lanef >= jnp.full((16,), lo2))
                          & (lanef <= jnp.full((16,), hi2)))
                    opv = jnp.where(km, jnp.full((16,), op), 0.0)
                    fv = [jnp.full((16,), pv1[cc]) if cc < 16
                          else jnp.full((16,), pv2[cc - 16])
                          for cc in range(NUM_CLASSES)]
                    j0 = lo1.astype(jnp.int32)
                    j1 = jnp.minimum(hi1, jnp.float32(W - 1)).astype(jnp.int32)

                    def j_body(j, cc):
                        jf = j.astype(jnp.float32)
                        yj = (jf + 0.5) * VOXEL_SIZE + VOL_LO[1]
                        dy = yj - my
                        s1 = dy * (ia11 * dy + c1)
                        s2 = 2.0 * ia12 * dy
                        maha = (v0 + jnp.full((16,), s1)
                                + jnp.full((16,), s2) * dzv)
                        d = opv * jnp.exp(-0.5 * maha)
                        base = j * 16
                        plsc.addupdate(buf.at[0, pl.ds(base, 16)], d)
                        for ch in range(NUM_CLASSES):
                            plsc.addupdate(
                                buf.at[ch + 1, pl.ds(base, 16)],
                                fv[ch] * d)
                        return cc

                    lax.fori_loop(j0, j1 + 1, j_body, 0)
                return c

            lax.fori_loop(0, GP, g_body, 0)

            def norm_body(j, c):
                base = j * 16
                dd = buf[0, pl.ds(base, 16)]
                r = 1.0 / jnp.maximum(dd, 1e-6)
                for ch in range(1, CH):
                    buf[ch, pl.ds(base, 16)] = buf[ch, pl.ds(base, 16)] * r
                return c

            lax.fori_loop(0, W, norm_body, 0)
            pltpu.sync_copy(buf, out_hbm.at[i])


@jax.jit
def kernel(means3d, opacities, features, scales, rotations, empty_scalar):
    H, W, D = GRID_SHAPE
    V = H * W * D
    N = means3d.shape[0]

    packed = jnp.zeros((16, GP), dtype=jnp.float32)
    packed = packed.at[0:4, :N].set(rotations.astype(jnp.float32).T)
    packed = packed.at[4:7, :N].set(scales.astype(jnp.float32).T)
    packed = packed.at[7:10, :N].set(means3d.astype(jnp.float32).T)
    packed = packed.at[10, :N].set(opacities.astype(jnp.float32)[:, 0])

    coefs = pl.pallas_call(
        _prologue_body,
        out_shape=jax.ShapeDtypeStruct((16, GP), jnp.float32),
    )(packed)

    params = jnp.concatenate(
        [coefs.T,
         features.astype(jnp.float32),
         jnp.zeros((N, 1), dtype=jnp.float32),
         jnp.zeros((N, PC - 16 - NUM_CLASSES), dtype=jnp.float32)],
        axis=1)
    params = params.at[:, 46].set(empty_scalar[0])

    mesh = plsc.VectorSubcoreMesh(core_axis_name="c", subcore_axis_name="s")
    o3 = pl.kernel(
        _sc_body,
        mesh=mesh,
        out_type=jax.ShapeDtypeStruct((H, CHP, W * D), jnp.float32),
        scratch_types=[
            pltpu.VMEM((GP * PC,), jnp.float32),
            pltpu.VMEM((CHP, W * D), jnp.float32),
            pltpu.VMEM((224,), jnp.float32),
        ],
    )(params.reshape(-1))

    grid_density = o3[:, 0, :].reshape(H, W, D, 1)
    grid_feats = o3[:, 1:CH, :].transpose(0, 2, 1).reshape(
        H, W, D, NUM_CLASSES)
    return grid_density, grid_feats
